# Initial kernel scaffold; baseline (speedup 1.0000x reference)
import jax
import jax.numpy as jnp
from jax.experimental import pallas as pl


def kernel(node_features, from_idx, to_idx, edge_features, msg_W, msg_b, mlp_W1, mlp_b1, mlp_W2, mlp_b2):
    raise NotImplementedError("write your pallas kernel here")


# speedup vs baseline: 3.5168x; 3.5168x over previous
import functools

import jax
import jax.numpy as jnp
from jax import lax
from jax.experimental import pallas as pl
from jax.experimental.pallas import tpu as pltpu
from jax.experimental.pallas import tpu_sc as plsc


def _proj_body(x_ref, w_ref, pf_ref, pt_ref):
    x = x_ref[...]
    d = x.shape[-1]
    pf_ref[...] = jnp.dot(x, w_ref[0:d, :], preferred_element_type=jnp.float32)
    pt_ref[...] = jnp.dot(x, w_ref[d : 2 * d, :], preferred_element_type=jnp.float32)


def _node_projections(node_features, msg_W):
    n, d = node_features.shape
    dout = msg_W.shape[1]
    blk = 2000
    grid = n // blk
    return pl.pallas_call(
        _proj_body,
        grid=(grid,),
        in_specs=[
            pl.BlockSpec((blk, d), lambda i: (i, 0)),
            pl.BlockSpec(msg_W.shape, lambda i: (0, 0)),
        ],
        out_specs=[
            pl.BlockSpec((blk, dout), lambda i: (i, 0)),
            pl.BlockSpec((blk, dout), lambda i: (i, 0)),
        ],
        out_shape=[
            jax.ShapeDtypeStruct((n, dout), jnp.float32),
            jax.ShapeDtypeStruct((n, dout), jnp.float32),
        ],
    )(node_features, msg_W)


def _edge_body(ef_ref, w_ref, b_ref, out_ref):
    de = ef_ref.shape[-1]
    w = w_ref[w_ref.shape[0] - de :, :]
    out_ref[...] = (
        jnp.dot(ef_ref[...], w, preferred_element_type=jnp.float32) + b_ref[...]
    )


def _edge_projection(edge_features, msg_W, msg_b):
    e, de = edge_features.shape
    dout = msg_W.shape[1]
    blk = 3200
    grid = e // blk
    return pl.pallas_call(
        _edge_body,
        grid=(grid,),
        in_specs=[
            pl.BlockSpec((blk, de), lambda i: (i, 0)),
            pl.BlockSpec(msg_W.shape, lambda i: (0, 0)),
            pl.BlockSpec((1, dout), lambda i: (0, 0)),
        ],
        out_specs=pl.BlockSpec((blk, dout), lambda i: (i, 0)),
        out_shape=jax.ShapeDtypeStruct((e, dout), jnp.float32),
    )(edge_features, msg_W, msg_b.reshape(1, dout))


def _mlp_body(agg_ref, x_ref, w1_ref, b1_ref, w2_ref, b2_ref, out_ref):
    agg = agg_ref[0] + agg_ref[1]
    x = x_ref[...]
    d = x.shape[-1]
    h = jnp.maximum(
        jnp.dot(agg, w1_ref[0:d, :], preferred_element_type=jnp.float32)
        + jnp.dot(x, w1_ref[d : 2 * d, :], preferred_element_type=jnp.float32)
        + b1_ref[...],
        0.0,
    )
    h = jnp.maximum(
        jnp.dot(h, w2_ref[...], preferred_element_type=jnp.float32) + b2_ref[...],
        0.0,
    )
    out_ref[...] = x + h


def _node_update(agg_partials, node_features, mlp_W1, mlp_b1, mlp_W2, mlp_b2):
    n, d = node_features.shape
    blk = 2000
    grid = n // blk
    return pl.pallas_call(
        _mlp_body,
        grid=(grid,),
        in_specs=[
            pl.BlockSpec((2, blk, d), lambda i: (0, i, 0)),
            pl.BlockSpec((blk, d), lambda i: (i, 0)),
            pl.BlockSpec(mlp_W1.shape, lambda i: (0, 0)),
            pl.BlockSpec((1, d), lambda i: (0, 0)),
            pl.BlockSpec(mlp_W2.shape, lambda i: (0, 0)),
            pl.BlockSpec((1, d), lambda i: (0, 0)),
        ],
        out_specs=pl.BlockSpec((blk, d), lambda i: (i, 0)),
        out_shape=jax.ShapeDtypeStruct((n, d), jnp.float32),
    )(
        agg_partials,
        node_features,
        mlp_W1,
        mlp_b1.reshape(1, d),
        mlp_W2,
        mlp_b2.reshape(1, d),
    )


_NC = 2
_NS = 16
_NW = _NC * _NS
_B = 80
_L = 16


def _sc_body(
    pf_hbm,
    pt_hbm,
    ep_hbm,
    fidx_hbm,
    tidx_hbm,
    zeros_hbm,
    out_hbm,
    acc_sh,
    fidx_v,
    tidx_v,
    fr_v,
    tr_v,
    ep_v,
    sem_f,
    sem_t,
    sem_e,
):
    d = pf_hbm.shape[1]
    n_pad = zeros_hbm.shape[0]
    e = fidx_hbm.shape[0]
    ept = e // _NW
    nblocks = ept // _B
    rows = n_pad // _NS

    cid = lax.axis_index("c")
    sid = lax.axis_index("s")
    wid = sid * _NC + cid

    row0 = sid * rows
    pltpu.sync_copy(zeros_hbm.at[pl.ds(row0, rows), :], acc_sh.at[pl.ds(row0, rows), :])
    plsc.subcore_barrier()

    base0 = wid * ept

    def block(i, carry):
        base = base0 + i * _B
        pltpu.sync_copy(fidx_hbm.at[pl.ds(base, _B)], fidx_v)
        pltpu.sync_copy(tidx_hbm.at[pl.ds(base, _B)], tidx_v)
        cp_f = pltpu.async_copy(pf_hbm.at[fidx_v], fr_v, sem_f)
        cp_t = pltpu.async_copy(pt_hbm.at[tidx_v], tr_v, sem_t)
        cp_e = pltpu.async_copy(ep_hbm.at[pl.ds(base, _B), :], ep_v, sem_e)
        cp_f.wait()
        cp_t.wait()
        cp_e.wait()

        def row(r, c2):
            for cc in range(d // _L):
                s = pl.ds(cc * _L, _L)
                m = fr_v[r, s] + tr_v[r, s] + ep_v[r, s]
                fr_v[r, s] = jnp.maximum(m, 0.0)
            return c2

        lax.fori_loop(0, _B, row, 0)
        pltpu.sync_copy(fr_v, acc_sh.at[tidx_v], add=True)
        return carry

    lax.fori_loop(0, nblocks, block, 0)

    plsc.subcore_barrier()
    pltpu.sync_copy(
        acc_sh.at[pl.ds(row0, rows), :], out_hbm.at[cid, pl.ds(row0, rows), :]
    )


def _sc_aggregate(p_from, p_to, eproj, from_idx, to_idx, zeros):
    d = p_from.shape[1]
    n_pad = zeros.shape[0]
    mesh = plsc.VectorSubcoreMesh(core_axis_name="c", subcore_axis_name="s")
    kern = functools.partial(
        pl.kernel,
        out_type=jax.ShapeDtypeStruct((_NC, n_pad, d), jnp.float32),
        mesh=mesh,
        scratch_types=[
            pltpu.VMEM_SHARED((n_pad, d), jnp.float32),
            pltpu.VMEM((_B,), jnp.int32),
            pltpu.VMEM((_B,), jnp.int32),
            pltpu.VMEM((_B, d), jnp.float32),
            pltpu.VMEM((_B, d), jnp.float32),
            pltpu.VMEM((_B, d), jnp.float32),
            pltpu.SemaphoreType.DMA,
            pltpu.SemaphoreType.DMA,
            pltpu.SemaphoreType.DMA,
        ],
    )(_sc_body)
    return kern(p_from, p_to, eproj, from_idx, to_idx, zeros)


def kernel(
    node_features,
    from_idx,
    to_idx,
    edge_features,
    msg_W,
    msg_b,
    mlp_W1,
    mlp_b1,
    mlp_W2,
    mlp_b2,
):
    n, d = node_features.shape
    p_from, p_to = _node_projections(node_features, msg_W)
    eproj = _edge_projection(edge_features, msg_W, msg_b)
    n_pad = -(-n // (8 * _NS)) * (8 * _NS)
    zeros = jnp.zeros((n_pad, d), jnp.float32)
    agg_partials = _sc_aggregate(p_from, p_to, eproj, from_idx, to_idx, zeros)
    return _node_update(agg_partials, node_features, mlp_W1, mlp_b1, mlp_W2, mlp_b2)


# speedup vs baseline: 4.0840x; 1.1613x over previous
import functools

import jax
import jax.numpy as jnp
from jax import lax
from jax.experimental import pallas as pl
from jax.experimental.pallas import tpu as pltpu
from jax.experimental.pallas import tpu_sc as plsc


def _proj_body(x_ref, w_ref, pf_ref, pt_ref):
    x = x_ref[...]
    d = x.shape[-1]
    pf_ref[...] = jnp.dot(x, w_ref[0:d, :], preferred_element_type=jnp.float32)
    pt_ref[...] = jnp.dot(x, w_ref[d : 2 * d, :], preferred_element_type=jnp.float32)


def _node_projections(node_features, msg_W):
    n, d = node_features.shape
    dout = msg_W.shape[1]
    blk = 2000
    grid = n // blk
    return pl.pallas_call(
        _proj_body,
        grid=(grid,),
        in_specs=[
            pl.BlockSpec((blk, d), lambda i: (i, 0)),
            pl.BlockSpec(msg_W.shape, lambda i: (0, 0)),
        ],
        out_specs=[
            pl.BlockSpec((blk, dout), lambda i: (i, 0)),
            pl.BlockSpec((blk, dout), lambda i: (i, 0)),
        ],
        out_shape=[
            jax.ShapeDtypeStruct((n, dout), jnp.float32),
            jax.ShapeDtypeStruct((n, dout), jnp.float32),
        ],
    )(node_features, msg_W)


def _edge_body(ef_ref, w_ref, b_ref, out_ref):
    de = ef_ref.shape[-1]
    w = w_ref[w_ref.shape[0] - de :, :]
    out_ref[...] = (
        jnp.dot(ef_ref[...], w, preferred_element_type=jnp.float32) + b_ref[...]
    )


def _edge_projection(edge_features, msg_W, msg_b):
    e, de = edge_features.shape
    dout = msg_W.shape[1]
    blk = 3200
    grid = e // blk
    return pl.pallas_call(
        _edge_body,
        grid=(grid,),
        in_specs=[
            pl.BlockSpec((blk, de), lambda i: (i, 0)),
            pl.BlockSpec(msg_W.shape, lambda i: (0, 0)),
            pl.BlockSpec((1, dout), lambda i: (0, 0)),
        ],
        out_specs=pl.BlockSpec((blk, dout), lambda i: (i, 0)),
        out_shape=jax.ShapeDtypeStruct((e, dout), jnp.float32),
    )(edge_features, msg_W, msg_b.reshape(1, dout))


def _mlp_body(agg_ref, x_ref, w1_ref, b1_ref, w2_ref, b2_ref, out_ref):
    agg = agg_ref[0] + agg_ref[1]
    x = x_ref[...]
    d = x.shape[-1]
    h = jnp.maximum(
        jnp.dot(agg, w1_ref[0:d, :], preferred_element_type=jnp.float32)
        + jnp.dot(x, w1_ref[d : 2 * d, :], preferred_element_type=jnp.float32)
        + b1_ref[...],
        0.0,
    )
    h = jnp.maximum(
        jnp.dot(h, w2_ref[...], preferred_element_type=jnp.float32) + b2_ref[...],
        0.0,
    )
    out_ref[...] = x + h


def _node_update(agg_partials, node_features, mlp_W1, mlp_b1, mlp_W2, mlp_b2):
    n, d = node_features.shape
    blk = 2000
    grid = n // blk
    return pl.pallas_call(
        _mlp_body,
        grid=(grid,),
        in_specs=[
            pl.BlockSpec((2, blk, d), lambda i: (0, i, 0)),
            pl.BlockSpec((blk, d), lambda i: (i, 0)),
            pl.BlockSpec(mlp_W1.shape, lambda i: (0, 0)),
            pl.BlockSpec((1, d), lambda i: (0, 0)),
            pl.BlockSpec(mlp_W2.shape, lambda i: (0, 0)),
            pl.BlockSpec((1, d), lambda i: (0, 0)),
        ],
        out_specs=pl.BlockSpec((blk, d), lambda i: (i, 0)),
        out_shape=jax.ShapeDtypeStruct((n, d), jnp.float32),
    )(
        agg_partials,
        node_features,
        mlp_W1,
        mlp_b1.reshape(1, d),
        mlp_W2,
        mlp_b2.reshape(1, d),
    )


_NC = 2
_NS = 16
_NW = _NC * _NS
_B = 40
_CHUNK = 2000
_L = 16


def _sc_body(
    pf_hbm,
    pt_hbm,
    ep_hbm,
    fidx_hbm,
    tidx_hbm,
    zeros_hbm,
    out_hbm,
    acc_sh,
    fidx0,
    tidx0,
    fr0,
    tr0,
    ep0,
    fidx1,
    tidx1,
    fr1,
    tr1,
    ep1,
    semf0,
    semt0,
    seme0,
    semf1,
    semt1,
    seme1,
):
    d = pf_hbm.shape[1]
    n_pad = zeros_hbm.shape[0]
    e = fidx_hbm.shape[0]
    ept = e // _NW
    nblocks = ept // _B
    rows = n_pad // _NS

    cid = lax.axis_index("c")
    sid = lax.axis_index("s")
    wid = sid * _NC + cid

    row0 = sid * rows
    pltpu.sync_copy(zeros_hbm.at[pl.ds(row0, rows), :], acc_sh.at[pl.ds(row0, rows), :])
    plsc.subcore_barrier()

    base0 = wid * ept

    slot0 = (fidx0, tidx0, fr0, tr0, ep0, semf0, semt0, seme0)
    slot1 = (fidx1, tidx1, fr1, tr1, ep1, semf1, semt1, seme1)

    def issue(i, slot):
        fidx, tidx, fr, tr, ep, semf, semt, seme = slot
        base = base0 + i * _B
        pltpu.sync_copy(fidx_hbm.at[pl.ds(base, _B)], fidx)
        pltpu.sync_copy(tidx_hbm.at[pl.ds(base, _B)], tidx)
        pltpu.async_copy(pf_hbm.at[fidx], fr, semf)
        pltpu.async_copy(pt_hbm.at[tidx], tr, semt)
        pltpu.async_copy(ep_hbm.at[pl.ds(base, _B), :], ep, seme)

    def process(i, slot):
        fidx, tidx, fr, tr, ep, semf, semt, seme = slot
        pltpu.make_async_copy(pf_hbm.at[fidx], fr, semf).wait()
        pltpu.make_async_copy(pt_hbm.at[tidx], tr, semt).wait()
        base = base0 + i * _B
        pltpu.make_async_copy(ep_hbm.at[pl.ds(base, _B), :], ep, seme).wait()

        def row(r, c2):
            for cc in range(d // _L):
                s = pl.ds(cc * _L, _L)
                m = fr[r, s] + tr[r, s] + ep[r, s]
                fr[r, s] = jnp.maximum(m, 0.0)
            return c2

        lax.fori_loop(0, _B, row, 0)
        pltpu.sync_copy(fr, acc_sh.at[tidx], add=True)

    issue(0, slot0)

    def pair(g, c2):
        i = 2 * g
        issue(i + 1, slot1)
        process(i, slot0)
        issue(i + 2, slot0)
        process(i + 1, slot1)
        return c2

    lax.fori_loop(0, (nblocks - 2) // 2, pair, 0)

    issue(nblocks - 1, slot1)
    process(nblocks - 2, slot0)
    process(nblocks - 1, slot1)

    plsc.subcore_barrier()
    pltpu.sync_copy(
        acc_sh.at[pl.ds(row0, rows), :], out_hbm.at[cid, pl.ds(row0, rows), :]
    )


def _sc_aggregate(p_from, p_to, eproj, from_idx, to_idx, zeros):
    d = p_from.shape[1]
    n_pad = zeros.shape[0]
    mesh = plsc.VectorSubcoreMesh(core_axis_name="c", subcore_axis_name="s")
    kern = functools.partial(
        pl.kernel,
        out_type=jax.ShapeDtypeStruct((_NC, n_pad, d), jnp.float32),
        mesh=mesh,
        scratch_types=[
            pltpu.VMEM_SHARED((n_pad, d), jnp.float32),
            pltpu.VMEM((_B,), jnp.int32),
            pltpu.VMEM((_B,), jnp.int32),
            pltpu.VMEM((_B, d), jnp.float32),
            pltpu.VMEM((_B, d), jnp.float32),
            pltpu.VMEM((_B, d), jnp.float32),
            pltpu.VMEM((_B,), jnp.int32),
            pltpu.VMEM((_B,), jnp.int32),
            pltpu.VMEM((_B, d), jnp.float32),
            pltpu.VMEM((_B, d), jnp.float32),
            pltpu.VMEM((_B, d), jnp.float32),
            pltpu.SemaphoreType.DMA,
            pltpu.SemaphoreType.DMA,
            pltpu.SemaphoreType.DMA,
            pltpu.SemaphoreType.DMA,
            pltpu.SemaphoreType.DMA,
            pltpu.SemaphoreType.DMA,
        ],
    )(_sc_body)
    return kern(p_from, p_to, eproj, from_idx, to_idx, zeros)


def kernel(
    node_features,
    from_idx,
    to_idx,
    edge_features,
    msg_W,
    msg_b,
    mlp_W1,
    mlp_b1,
    mlp_W2,
    mlp_b2,
):
    n, d = node_features.shape
    p_from, p_to = _node_projections(node_features, msg_W)
    eproj = _edge_projection(edge_features, msg_W, msg_b)
    n_pad = -(-n // (8 * _NS)) * (8 * _NS)
    zeros = jnp.zeros((n_pad, d), jnp.float32)
    agg_partials = _sc_aggregate(p_from, p_to, eproj, from_idx, to_idx, zeros)
    return _node_update(agg_partials, node_features, mlp_W1, mlp_b1, mlp_W2, mlp_b2)
